# Initial kernel scaffold; baseline (speedup 1.0000x reference)
import jax
import jax.numpy as jnp
from jax.experimental import pallas as pl


def kernel(x, pos_table):
    raise NotImplementedError("write your pallas kernel here")


# speedup vs baseline: 3.2246x; 3.2246x over previous
import jax
import jax.numpy as jnp
from jax.experimental import pallas as pl


_L_BLOCK = 256


def _body(x_ref, p_ref, o_ref):
    o_ref[...] = x_ref[...] + p_ref[...][None]


def kernel(x, pos_table):
    B, L, D = x.shape
    lb = _L_BLOCK
    grid = (L // lb,)
    return pl.pallas_call(
        _body,
        grid=grid,
        in_specs=[
            pl.BlockSpec((B, lb, D), lambda i: (0, i, 0)),
            pl.BlockSpec((lb, D), lambda i: (i, 0)),
        ],
        out_specs=pl.BlockSpec((B, lb, D), lambda i: (0, i, 0)),
        out_shape=jax.ShapeDtypeStruct((B, L, D), x.dtype),
    )(x, pos_table[:L])


# speedup vs baseline: 3.2911x; 1.0206x over previous
import jax
import jax.numpy as jnp
from jax.experimental import pallas as pl


_L_BLOCK = 512


def _body(x_ref, p_ref, o_ref):
    o_ref[...] = x_ref[...] + p_ref[...][None]


def kernel(x, pos_table):
    B, L, D = x.shape
    lb = _L_BLOCK
    grid = (L // lb,)
    return pl.pallas_call(
        _body,
        grid=grid,
        in_specs=[
            pl.BlockSpec((B, lb, D), lambda i: (0, i, 0)),
            pl.BlockSpec((lb, D), lambda i: (i, 0)),
        ],
        out_specs=pl.BlockSpec((B, lb, D), lambda i: (0, i, 0)),
        out_shape=jax.ShapeDtypeStruct((B, L, D), x.dtype),
    )(x, pos_table[:L])
